# baseline (speedup 1.0000x reference)
import functools

import jax
import jax.numpy as jnp
from jax.experimental import pallas as pl

IMAGE_SIZE = 512
PATCH = 16
CODEBOOK_SIZE = 8192
CODE_DIM = 32
IN_CH = 3

N_TILE = 512
K_CHUNK = 2048


def _vq_kernel(p_ref, w_ref, cb_ref, c2_ref, out_ref):
    pb = p_ref[...].astype(jnp.bfloat16)
    wb = w_ref[...].astype(jnp.bfloat16)
    z = jnp.dot(pb, wb, preferred_element_type=jnp.float32)
    z2 = jnp.sum(z * z, axis=1, keepdims=True)
    zb = z.astype(jnp.bfloat16)

    run_min = jnp.full((N_TILE, 1), jnp.inf, dtype=jnp.float32)
    run_arg = jnp.zeros((N_TILE, 1), dtype=jnp.int32)

    dn = (((1,), (1,)), ((), ()))
    for c in range(CODEBOOK_SIZE // K_CHUNK):
        cb = cb_ref[pl.ds(c * K_CHUNK, K_CHUNK), :].astype(jnp.bfloat16)
        s = jax.lax.dot_general(zb, cb, dn, preferred_element_type=jnp.float32)
        c2 = c2_ref[:, pl.ds(c * K_CHUNK, K_CHUNK)]
        d = (z2 - 2.0 * s) + c2
        cmin = jnp.min(d, axis=1, keepdims=True)
        idx = jax.lax.broadcasted_iota(jnp.int32, (N_TILE, K_CHUNK), 1)
        cand = jnp.where(d == cmin, idx, CODEBOOK_SIZE)
        carg = jnp.min(cand, axis=1, keepdims=True) + c * K_CHUNK
        better = cmin < run_min
        run_min = jnp.where(better, cmin, run_min)
        run_arg = jnp.where(better, carg, run_arg)

    out_ref[...] = run_arg


@functools.partial(jax.jit, static_argnames=())
def _encode(x, W, codebook):
    B = x.shape[0]
    h = IMAGE_SIZE // PATCH
    w = IMAGE_SIZE // PATCH
    n_total = B * h * w
    patches = (
        x.reshape(B, IN_CH, h, PATCH, w, PATCH)
        .transpose(0, 2, 4, 1, 3, 5)
        .reshape(n_total, IN_CH * PATCH * PATCH)
    )
    c2 = jnp.sum(codebook * codebook, axis=-1)[None, :]
    grid = (n_total // N_TILE,)
    out = pl.pallas_call(
        _vq_kernel,
        grid=grid,
        in_specs=[
            pl.BlockSpec((N_TILE, IN_CH * PATCH * PATCH), lambda i: (i, 0)),
            pl.BlockSpec((IN_CH * PATCH * PATCH, CODE_DIM), lambda i: (0, 0)),
            pl.BlockSpec((CODEBOOK_SIZE, CODE_DIM), lambda i: (0, 0)),
            pl.BlockSpec((1, CODEBOOK_SIZE), lambda i: (0, 0)),
        ],
        out_specs=pl.BlockSpec((N_TILE, 1), lambda i: (i, 0)),
        out_shape=jax.ShapeDtypeStruct((n_total, 1), jnp.int32),
    )(patches, W, codebook, c2)
    return out.reshape(B, h * w)


def kernel(x, W, codebook):
    indices = _encode(x, W, codebook)
    return (indices, IMAGE_SIZE // PATCH, IMAGE_SIZE // PATCH)


# speedup vs baseline: 1.0090x; 1.0090x over previous
import functools

import jax
import jax.numpy as jnp
from jax.experimental import pallas as pl

IMAGE_SIZE = 512
PATCH = 16
CODEBOOK_SIZE = 8192
CODE_DIM = 32
IN_CH = 3

N_TILE = 512
K_CHUNK = 2048


def _vq_kernel(p_ref, w_ref, cb_ref, c2_ref, out_ref):
    pb = p_ref[...].astype(jnp.bfloat16)
    wb = w_ref[...].astype(jnp.bfloat16)
    z = jnp.dot(pb, wb, preferred_element_type=jnp.float32)
    z2 = jnp.sum(z * z, axis=1, keepdims=True)
    zb2 = (2.0 * z).astype(jnp.bfloat16)

    run_min = jnp.full((N_TILE, 1), jnp.inf, dtype=jnp.float32)
    run_arg = jnp.zeros((N_TILE, 1), dtype=jnp.int32)
    idx = jax.lax.broadcasted_iota(jnp.int32, (N_TILE, K_CHUNK), 1)

    dn = (((1,), (1,)), ((), ()))
    for c in range(CODEBOOK_SIZE // K_CHUNK):
        cb = cb_ref[pl.ds(c * K_CHUNK, K_CHUNK), :].astype(jnp.bfloat16)
        s2 = jax.lax.dot_general(zb2, cb, dn, preferred_element_type=jnp.float32)
        c2 = c2_ref[:, pl.ds(c * K_CHUNK, K_CHUNK)]
        d = (z2 - s2) + c2
        cmin = jnp.min(d, axis=1, keepdims=True)
        cand = jnp.where(d == cmin, idx, CODEBOOK_SIZE)
        carg = jnp.min(cand, axis=1, keepdims=True) + c * K_CHUNK
        better = cmin < run_min
        run_min = jnp.where(better, cmin, run_min)
        run_arg = jnp.where(better, carg, run_arg)

    out_ref[...] = run_arg


@functools.partial(jax.jit, static_argnames=())
def _encode(x, W, codebook):
    B = x.shape[0]
    h = IMAGE_SIZE // PATCH
    w = IMAGE_SIZE // PATCH
    n_total = B * h * w
    patches = (
        x.reshape(B, IN_CH, h, PATCH, w, PATCH)
        .transpose(0, 2, 4, 1, 3, 5)
        .reshape(n_total, IN_CH * PATCH * PATCH)
    )
    c2 = jnp.sum(codebook * codebook, axis=-1)[None, :]
    grid = (n_total // N_TILE,)
    out = pl.pallas_call(
        _vq_kernel,
        grid=grid,
        in_specs=[
            pl.BlockSpec((N_TILE, IN_CH * PATCH * PATCH), lambda i: (i, 0)),
            pl.BlockSpec((IN_CH * PATCH * PATCH, CODE_DIM), lambda i: (0, 0)),
            pl.BlockSpec((CODEBOOK_SIZE, CODE_DIM), lambda i: (0, 0)),
            pl.BlockSpec((1, CODEBOOK_SIZE), lambda i: (0, 0)),
        ],
        out_specs=pl.BlockSpec((N_TILE, 1), lambda i: (i, 0)),
        out_shape=jax.ShapeDtypeStruct((n_total, 1), jnp.int32),
    )(patches, W, codebook, c2)
    return out.reshape(B, h * w)


def kernel(x, W, codebook):
    indices = _encode(x, W, codebook)
    return (indices, IMAGE_SIZE // PATCH, IMAGE_SIZE // PATCH)


# speedup vs baseline: 1.0800x; 1.0704x over previous
import functools

import jax
import jax.numpy as jnp
from jax.experimental import pallas as pl

IMAGE_SIZE = 512
PATCH = 16
CODEBOOK_SIZE = 8192
CODE_DIM = 32
IN_CH = 3

N_TILE = 512
K_CHUNK = 2048


def _vq_kernel(p_ref, w_ref, cb_ref, c2_ref, out_ref):
    pb = p_ref[...].astype(jnp.bfloat16)
    wb = w_ref[...].astype(jnp.bfloat16)
    z = jnp.dot(pb, wb, preferred_element_type=jnp.float32)
    z2 = jnp.sum(z * z, axis=1, keepdims=True)
    zb2 = (2.0 * z).astype(jnp.bfloat16)

    dn = (((1,), (1,)), ((), ()))
    cb = cb_ref[...].astype(jnp.bfloat16)
    s2 = jax.lax.dot_general(zb2, cb, dn, preferred_element_type=jnp.float32)
    d = (z2 - s2) + c2_ref[...]
    out_ref[...] = jnp.argmin(d, axis=1)[:, None]


@functools.partial(jax.jit, static_argnames=())
def _encode(x, W, codebook):
    B = x.shape[0]
    h = IMAGE_SIZE // PATCH
    w = IMAGE_SIZE // PATCH
    n_total = B * h * w
    patches = (
        x.reshape(B, IN_CH, h, PATCH, w, PATCH)
        .transpose(0, 2, 4, 1, 3, 5)
        .reshape(n_total, IN_CH * PATCH * PATCH)
    )
    c2 = jnp.sum(codebook * codebook, axis=-1)[None, :]
    grid = (n_total // N_TILE,)
    out = pl.pallas_call(
        _vq_kernel,
        grid=grid,
        in_specs=[
            pl.BlockSpec((N_TILE, IN_CH * PATCH * PATCH), lambda i: (i, 0)),
            pl.BlockSpec((IN_CH * PATCH * PATCH, CODE_DIM), lambda i: (0, 0)),
            pl.BlockSpec((CODEBOOK_SIZE, CODE_DIM), lambda i: (0, 0)),
            pl.BlockSpec((1, CODEBOOK_SIZE), lambda i: (0, 0)),
        ],
        out_specs=pl.BlockSpec((N_TILE, 1), lambda i: (i, 0)),
        out_shape=jax.ShapeDtypeStruct((n_total, 1), jnp.int32),
    )(patches, W, codebook, c2)
    return out.reshape(B, h * w)


def kernel(x, W, codebook):
    indices = _encode(x, W, codebook)
    return (indices, IMAGE_SIZE // PATCH, IMAGE_SIZE // PATCH)


# speedup vs baseline: 1.6017x; 1.4830x over previous
import functools

import jax
import jax.numpy as jnp
from jax.experimental import pallas as pl

IMAGE_SIZE = 512
PATCH = 16
CODEBOOK_SIZE = 8192
CODE_DIM = 32
IN_CH = 3

GRID_H = IMAGE_SIZE // PATCH
GRID_W = IMAGE_SIZE // PATCH
PH_TILE = 16
N_TILE = PH_TILE * GRID_W
FEAT = IN_CH * PATCH * PATCH


def _vq_kernel(x_ref, w_ref, cb_ref, c2_ref, out_ref):
    v = x_ref[0]
    pieces = []
    for c in range(IN_CH):
        for i in range(PATCH):
            pieces.append(v[c, :, i, :, :])
    patches = jnp.concatenate(pieces, axis=-1).reshape(N_TILE, FEAT)

    pb = patches.astype(jnp.bfloat16)
    wb = w_ref[...].astype(jnp.bfloat16)
    z = jnp.dot(pb, wb, preferred_element_type=jnp.float32)
    z2 = jnp.sum(z * z, axis=1, keepdims=True)
    zb2 = (2.0 * z).astype(jnp.bfloat16)

    dn = (((1,), (1,)), ((), ()))
    cb = cb_ref[...].astype(jnp.bfloat16)
    s2 = jax.lax.dot_general(zb2, cb, dn, preferred_element_type=jnp.float32)
    d = (z2 - s2) + c2_ref[...]
    out_ref[...] = jnp.argmin(d, axis=1)[:, None]


@jax.jit
def _encode(x, W, codebook):
    B = x.shape[0]
    n_total = B * GRID_H * GRID_W
    xv = x.reshape(B, IN_CH, GRID_H, PATCH, GRID_W, PATCH)
    c2 = jnp.sum(codebook * codebook, axis=-1)[None, :]
    steps_per_img = GRID_H // PH_TILE
    grid = (B * steps_per_img,)
    out = pl.pallas_call(
        _vq_kernel,
        grid=grid,
        in_specs=[
            pl.BlockSpec(
                (1, IN_CH, PH_TILE, PATCH, GRID_W, PATCH),
                lambda i: (i // steps_per_img, 0, i % steps_per_img, 0, 0, 0),
            ),
            pl.BlockSpec((FEAT, CODE_DIM), lambda i: (0, 0)),
            pl.BlockSpec((CODEBOOK_SIZE, CODE_DIM), lambda i: (0, 0)),
            pl.BlockSpec((1, CODEBOOK_SIZE), lambda i: (0, 0)),
        ],
        out_specs=pl.BlockSpec((N_TILE, 1), lambda i: (i, 0)),
        out_shape=jax.ShapeDtypeStruct((n_total, 1), jnp.int32),
    )(xv, W, codebook, c2)
    return out.reshape(B, GRID_H * GRID_W)


def kernel(x, W, codebook):
    indices = _encode(x, W, codebook)
    return (indices, GRID_H, GRID_W)
